# baseline (speedup 1.0000x reference)
import functools

import jax
import jax.numpy as jnp
from jax import lax
from jax.experimental import pallas as pl
from jax.experimental.pallas import tpu as pltpu
from jax.experimental.pallas import tpu_sc as plsc

_NC = 2
_NS = 16
_NW = _NC * _NS
_LANES = 16


@functools.partial(jax.jit, static_argnums=(2, 3, 4))
def _sc_pos_add(x_flat, pos_flat, B, T, D):
    t_per = T // _NW
    CH = 16
    n_ch = t_per // CH
    chunk = CH * D
    n_vec = chunk // _LANES
    UNROLL = 8

    mesh = plsc.VectorSubcoreMesh(core_axis_name="c", subcore_axis_name="s")

    @functools.partial(
        pl.kernel,
        mesh=mesh,
        out_type=jax.ShapeDtypeStruct((B * T * D,), jnp.float32),
        scratch_types=[
            pltpu.VMEM((chunk,), jnp.float32),
            pltpu.VMEM((chunk,), jnp.float32),
        ],
    )
    def k(x_hbm, pos_hbm, out_hbm, pos_v, x_v):
        w = lax.axis_index("s") * _NC + lax.axis_index("c")
        t0 = w * t_per

        def chunk_body(c, _):
            row0 = t0 + c * CH
            pltpu.sync_copy(pos_hbm.at[pl.ds(row0 * D, chunk)], pos_v)

            def b_body(b, _):
                xoff = (b * T + row0) * D
                pltpu.sync_copy(x_hbm.at[pl.ds(xoff, chunk)], x_v)

                def add_body(i, _):
                    base = i * (_LANES * UNROLL)
                    for u in range(UNROLL):
                        o = base + u * _LANES
                        x_v[pl.ds(o, _LANES)] = (
                            x_v[pl.ds(o, _LANES)] + pos_v[pl.ds(o, _LANES)]
                        )
                    return 0

                lax.fori_loop(0, n_vec // UNROLL, add_body, 0, unroll=False)
                pltpu.sync_copy(x_v, out_hbm.at[pl.ds(xoff, chunk)])
                return 0

            lax.fori_loop(0, B, b_body, 0, unroll=False)
            return 0

        lax.fori_loop(0, n_ch, chunk_body, 0, unroll=False)

    return k(x_flat, pos_flat)


def kernel(x, pos_weight):
    B, T, D = x.shape
    out_flat = _sc_pos_add(
        x.reshape(B * T * D), pos_weight.reshape(T * D), B, T, D
    )
    return out_flat.reshape(B, T, D)


# speedup vs baseline: 1.2711x; 1.2711x over previous
import functools

import jax
import jax.numpy as jnp
from jax import lax
from jax.experimental import pallas as pl
from jax.experimental.pallas import tpu as pltpu
from jax.experimental.pallas import tpu_sc as plsc

_NC = 2
_NS = 16
_NW = _NC * _NS
_LANES = 16


@functools.partial(jax.jit, static_argnums=(2, 3, 4))
def _sc_pos_add(x_flat, pos_flat, B, T, D):
    t_per = T // _NW
    CH = 16
    n_ch = t_per // CH
    chunk = CH * D
    n_vec = chunk // _LANES
    UNROLL = 8
    n_steps = n_ch * B
    n_g = n_steps // 2

    mesh = plsc.VectorSubcoreMesh(core_axis_name="c", subcore_axis_name="s")

    @functools.partial(
        pl.kernel,
        mesh=mesh,
        out_type=jax.ShapeDtypeStruct((B * T * D,), jnp.float32),
        scratch_types=[
            pltpu.VMEM((chunk,), jnp.float32),
            pltpu.VMEM((chunk,), jnp.float32),
            pltpu.VMEM((chunk,), jnp.float32),
            pltpu.VMEM((chunk,), jnp.float32),
            pltpu.VMEM((chunk,), jnp.float32),
            pltpu.SemaphoreType.DMA,
            pltpu.SemaphoreType.DMA,
            pltpu.SemaphoreType.DMA,
            pltpu.SemaphoreType.DMA,
        ],
    )
    def k(x_hbm, pos_hbm, out_hbm, pos_v, x0, x1, o0, o1, sx0, sx1, so0, so1):
        w = lax.axis_index("s") * _NC + lax.axis_index("c")
        t0 = w * t_per
        xbufs, obufs = (x0, x1), (o0, o1)
        sxs, sos = (sx0, sx1), (so0, so1)

        def xoff(s):
            c = s // B
            b = s - c * B
            return (b * T + t0 + c * CH) * D

        pltpu.async_copy(x_hbm.at[pl.ds(xoff(0), chunk)], x0, sx0)
        pltpu.async_copy(x_hbm.at[pl.ds(xoff(1), chunk)], x1, sx1)

        def g_body(g, _):
            for u in (0, 1):
                s = g * 2 + u
                c = s // B
                b = s - c * B
                xv, ov, sx, so = xbufs[u], obufs[u], sxs[u], sos[u]

                @pl.when(b == 0)
                def _():
                    pltpu.sync_copy(
                        pos_hbm.at[pl.ds((t0 + c * CH) * D, chunk)], pos_v
                    )

                pltpu.make_async_copy(x_hbm.at[pl.ds(0, chunk)], xv, sx).wait()

                @pl.when(g > 0)
                def _():
                    pltpu.make_async_copy(
                        ov, out_hbm.at[pl.ds(0, chunk)], so
                    ).wait()

                def add_body(i, _):
                    base = i * (_LANES * UNROLL)
                    for uu in range(UNROLL):
                        off = base + uu * _LANES
                        ov[pl.ds(off, _LANES)] = (
                            xv[pl.ds(off, _LANES)] + pos_v[pl.ds(off, _LANES)]
                        )
                    return 0

                lax.fori_loop(0, n_vec // UNROLL, add_body, 0, unroll=False)

                pltpu.async_copy(ov, out_hbm.at[pl.ds(xoff(s), chunk)], so)

                @pl.when(g < n_g - 1)
                def _():
                    pltpu.async_copy(
                        x_hbm.at[pl.ds(xoff(s + 2), chunk)], xv, sx
                    )
            return 0

        lax.fori_loop(0, n_g, g_body, 0, unroll=False)
        pltpu.make_async_copy(o0, out_hbm.at[pl.ds(0, chunk)], so0).wait()
        pltpu.make_async_copy(o1, out_hbm.at[pl.ds(0, chunk)], so1).wait()

    return k(x_flat, pos_flat)


def kernel(x, pos_weight):
    B, T, D = x.shape
    out_flat = _sc_pos_add(
        x.reshape(B * T * D), pos_weight.reshape(T * D), B, T, D
    )
    return out_flat.reshape(B, T, D)
